# Initial kernel scaffold; baseline (speedup 1.0000x reference)
import jax
import jax.numpy as jnp
from jax.experimental import pallas as pl


def kernel(x, encoding):
    raise NotImplementedError("write your pallas kernel here")


# speedup vs baseline: 2.1459x; 2.1459x over previous
import jax
import jax.numpy as jnp
from jax.experimental import pallas as pl


def _add_body(x_ref, enc_ref, out_ref):
    out_ref[...] = x_ref[...] + enc_ref[...][None]


def kernel(x, encoding):
    B, L, D = x.shape
    enc = encoding[:L]
    LB = 256
    grid = (L // LB,)
    return pl.pallas_call(
        _add_body,
        grid=grid,
        in_specs=[
            pl.BlockSpec((B, LB, D), lambda i: (0, i, 0)),
            pl.BlockSpec((LB, D), lambda i: (i, 0)),
        ],
        out_specs=pl.BlockSpec((B, LB, D), lambda i: (0, i, 0)),
        out_shape=jax.ShapeDtypeStruct((B, L, D), x.dtype),
    )(x, enc)
